# baseline (speedup 1.0000x reference)
import jax
import jax.numpy as jnp
from jax import lax
from jax.experimental import pallas as pl
from jax.experimental.pallas import tpu as pltpu
from jax.experimental.pallas import tpu_sc as plsc

B = 4096
H = 50
D = 128
C = 10
LANES = 16
DCH = D // LANES

NC = 2
NS = 16
NW = NC * NS
BPW = B // NW

NACC = 5
NGRP = H // NACC


def _pool_body(table_hbm, textt_hbm, out_hbm, idx_v, acc_v, *sems):
    wid = lax.axis_index("s") * NC + lax.axis_index("c")
    b0 = wid * BPW
    pltpu.sync_copy(textt_hbm.at[:, pl.ds(b0, BPW)], idx_v)

    def start(l, s, add):
        pltpu.async_copy(
            table_hbm.at[idx_v.at[l]], acc_v.at[s], sems[s], add=add)

    def wait(s):
        pltpu.make_async_copy(
            table_hbm.at[idx_v.at[0]], acc_v.at[s], sems[s]).wait()

    for s in range(NACC):
        start(s, s, False)

    def group(g, carry):
        for s in range(NACC):
            wait(s)
            start(g * NACC + s, s, True)
        return carry

    lax.fori_loop(1, NGRP, group, 0)
    for s in range(NACC):
        wait(s)

    def combine(j, carry):
        for c in range(DCH):
            sl = pl.ds(c * LANES, LANES)
            v = acc_v[0, j, sl]
            for s in range(1, NACC):
                v = v + acc_v[s, j, sl]
            acc_v[0, j, sl] = v
        return carry

    lax.fori_loop(0, BPW, combine, 0, unroll=4)
    pltpu.sync_copy(acc_v.at[0], out_hbm.at[pl.ds(b0, BPW)])


_pool = pl.kernel(
    _pool_body,
    out_type=jax.ShapeDtypeStruct((B, D), jnp.float32),
    mesh=plsc.VectorSubcoreMesh(core_axis_name="c", subcore_axis_name="s"),
    scratch_types=[
        pltpu.VMEM((H, BPW), jnp.int32),
        pltpu.VMEM((NACC, BPW, D), jnp.float32),
    ] + [pltpu.SemaphoreType.DMA] * NACC,
)


def _fc_body(x_ref, w_ref, b_ref, o_ref):
    o_ref[...] = jnp.dot(x_ref[...], w_ref[...],
                         preferred_element_type=jnp.float32) + b_ref[...]


def kernel(text, emb_table, fc_w, fc_b):
    textt = text.astype(jnp.int32).T
    pooled = _pool(emb_table, textt)
    wt = fc_w.T * jnp.float32(1.0 / H)
    out = pl.pallas_call(
        _fc_body,
        out_shape=jax.ShapeDtypeStruct((B, C), jnp.float32),
    )(pooled, wt, fc_b.reshape(1, C))
    return out
